# Initial kernel scaffold; baseline (speedup 1.0000x reference)
import jax
import jax.numpy as jnp
from jax.experimental import pallas as pl


def kernel(token_ids, table):
    raise NotImplementedError("write your pallas kernel here")


# speedup vs baseline: 4.5539x; 4.5539x over previous
import functools

import jax
import jax.numpy as jnp
from jax import lax
from jax.experimental import pallas as pl
from jax.experimental.pallas import tpu as pltpu
from jax.experimental.pallas import tpu_sc as plsc

PADDING_IDX = 0

NC = 2
NS = 16
NW = NC * NS
LANES = 16

SUB = 128
SUBS_PER_CHUNK = 8
CHUNK = SUB * SUBS_PER_CHUNK


def _embed_body(idx_hbm, table_hbm, out_hbm, idx_v, rows_v, gsem):
    D = table_hbm.shape[1]
    n_idx_rows = idx_hbm.shape[0]
    chunks_total = n_idx_rows // SUBS_PER_CHUNK
    chunks_per_w = chunks_total // NW

    wid = lax.axis_index("s") * NC + lax.axis_index("c")

    def chunk_body(g, _):
        c = wid * chunks_per_w + g
        pltpu.sync_copy(idx_hbm.at[pl.ds(c * SUBS_PER_CHUNK, SUBS_PER_CHUNK)],
                        idx_v)
        descs = [
            pltpu.async_copy(table_hbm.at[idx_v.at[j]],
                             rows_v.at[pl.ds(j * SUB, SUB)], gsem)
            for j in range(SUBS_PER_CHUNK)
        ]
        for d in descs:
            d.wait()

        def group_body(t, _):
            j = t // (SUB // LANES)
            k = t % (SUB // LANES)
            idx16 = idx_v[j, pl.ds(k * LANES, LANES)]
            m = idx16 == PADDING_IDX
            rowids = t * LANES + lax.iota(jnp.int32, LANES)
            zeros = jnp.zeros((LANES,), jnp.float32)
            for col in range(D):
                cols = jnp.full((LANES,), col, jnp.int32)
                plsc.store_scatter(rows_v, [rowids, cols], zeros, mask=m)
            return ()

        lax.fori_loop(0, CHUNK // LANES, group_body, (), unroll=False)

        pltpu.sync_copy(rows_v, out_hbm.at[pl.ds(c * CHUNK, CHUNK)])
        return ()

    lax.fori_loop(0, chunks_per_w, chunk_body, (), unroll=False)


@jax.jit
def _embed(idx2d, table):
    n_rows = idx2d.shape[0] * SUB
    D = table.shape[1]
    mesh = plsc.VectorSubcoreMesh(core_axis_name="c", subcore_axis_name="s")
    f = pl.kernel(
        _embed_body,
        out_type=jax.ShapeDtypeStruct((n_rows, D), jnp.float32),
        mesh=mesh,
        scratch_types=[
            pltpu.VMEM((SUBS_PER_CHUNK, SUB), jnp.int32),
            pltpu.VMEM((CHUNK, D), jnp.float32),
            pltpu.SemaphoreType.DMA,
        ],
        compiler_params=pltpu.CompilerParams(needs_layout_passes=False,
                                             use_tc_tiling_on_sc=False),
    )
    return f(idx2d, table)


def kernel(token_ids, table):
    B, H = token_ids.shape
    D = table.shape[1]
    idx2d = token_ids.reshape(-1, SUB).astype(jnp.int32)
    out = _embed(idx2d, table)
    return out.reshape(B, H, D)


# speedup vs baseline: 5.0752x; 1.1145x over previous
import jax
import jax.numpy as jnp
from jax import lax
from jax.experimental import pallas as pl
from jax.experimental.pallas import tpu as pltpu
from jax.experimental.pallas import tpu_sc as plsc

PADDING_IDX = 0

NC = 2
NS = 16
NW = NC * NS
LANES = 16

SUB = 128
SUBS = 4
CHUNK = SUB * SUBS
NBUF = 4


def _embed_body(idx_hbm, table_hbm, out_hbm, idx_v, rows_v, *sems):
    isems = sems[0:NBUF]
    gsems = sems[NBUF:2 * NBUF]
    osems = sems[2 * NBUF:3 * NBUF]
    D = table_hbm.shape[1]
    chunks_total = idx_hbm.shape[0] // SUBS
    n = chunks_total // NW
    wid = lax.axis_index("s") * NC + lax.axis_index("c")
    c0 = wid * n

    def fire_i(g, s):
        pltpu.async_copy(idx_hbm.at[pl.ds((c0 + g) * SUBS, SUBS)],
                         idx_v.at[s], isems[s])

    def wait_i(s):
        pltpu.make_async_copy(idx_hbm.at[pl.ds(0, SUBS)],
                              idx_v.at[s], isems[s]).wait()

    def fire_g(s):
        for j in range(SUBS):
            pltpu.async_copy(table_hbm.at[idx_v.at[s, j]],
                             rows_v.at[s, pl.ds(j * SUB, SUB)], gsems[s])

    def wait_g(s):
        pltpu.make_async_copy(out_hbm.at[pl.ds(0, CHUNK)],
                              rows_v.at[s], gsems[s]).wait()

    def fire_o(g, s):
        pltpu.async_copy(rows_v.at[s],
                         out_hbm.at[pl.ds((c0 + g) * CHUNK, CHUNK)], osems[s])

    def wait_o(s):
        pltpu.make_async_copy(rows_v.at[s],
                              out_hbm.at[pl.ds(0, CHUNK)], osems[s]).wait()

    def process(s):
        acc = jnp.full((LANES,), jnp.iinfo(jnp.int32).max, jnp.int32)
        for j in range(SUBS):
            for k in range(SUB // LANES):
                acc = jnp.minimum(acc, idx_v[s, j, pl.ds(k * LANES, LANES)])
        has_pad = jnp.min(acc) == PADDING_IDX

        @pl.when(has_pad)
        def _mask_pass():
            rows = rows_v.at[s]
            zeros = jnp.zeros((LANES,), jnp.float32)

            def group_body(t, _):
                j = t // (SUB // LANES)
                k = t % (SUB // LANES)
                idx16 = idx_v[s, j, pl.ds(k * LANES, LANES)]
                m = idx16 == PADDING_IDX
                rowids = t * LANES + lax.iota(jnp.int32, LANES)
                for col in range(D):
                    cols = jnp.full((LANES,), col, jnp.int32)
                    plsc.store_scatter(rows, [rowids, cols], zeros, mask=m)
                return ()

            lax.fori_loop(0, CHUNK // LANES, group_body, (), unroll=False)

    def chunk_iter(g, u, do_wait_o, do_fire_i, do_fire_g):
        if do_wait_o:
            wait_o((u + 2) % NBUF)
        if do_fire_i:
            fire_i(g + 3, (u + 3) % NBUF)
        if do_fire_g:
            wait_i((u + 2) % NBUF)
            fire_g((u + 2) % NBUF)
        wait_g(u)
        process(u)
        fire_o(g, u)

    fire_i(0, 0)
    fire_i(1, 1)
    wait_i(0)
    fire_g(0)
    wait_i(1)
    fire_g(1)
    fire_i(2, 2)

    for u in range(NBUF):
        chunk_iter(u, u, do_wait_o=(u >= 2), do_fire_i=True, do_fire_g=True)

    def outer_body(t, _):
        for u in range(NBUF):
            chunk_iter(t * NBUF + u, u, True, True, True)
        return ()

    lax.fori_loop(1, n // NBUF - 1, outer_body, (), unroll=False)

    for u in range(NBUF):
        g = n - NBUF + u
        chunk_iter(g, u, do_wait_o=True,
                   do_fire_i=(u + 3 < NBUF), do_fire_g=(u + 2 < NBUF))

    wait_o(2)
    wait_o(3)


@jax.jit
def _embed(idx2d, table):
    n_rows = idx2d.shape[0] * SUB
    D = table.shape[1]
    mesh = plsc.VectorSubcoreMesh(core_axis_name="c", subcore_axis_name="s")
    f = pl.kernel(
        _embed_body,
        out_type=jax.ShapeDtypeStruct((n_rows, D), jnp.float32),
        mesh=mesh,
        scratch_types=(
            [pltpu.VMEM((NBUF, SUBS, SUB), jnp.int32),
             pltpu.VMEM((NBUF, CHUNK, D), jnp.float32)]
            + [pltpu.SemaphoreType.DMA] * (3 * NBUF)
        ),
        compiler_params=pltpu.CompilerParams(needs_layout_passes=False,
                                             use_tc_tiling_on_sc=False),
    )
    return f(idx2d, table)


def kernel(token_ids, table):
    B, H = token_ids.shape
    D = table.shape[1]
    idx2d = token_ids.reshape(-1, SUB).astype(jnp.int32)
    out = _embed(idx2d, table)
    return out.reshape(B, H, D)
